# baseline (speedup 1.0000x reference)
import functools

import jax
import jax.numpy as jnp
from jax import lax
from jax.experimental import pallas as pl
from jax.experimental.pallas import tpu as pltpu
from jax.experimental.pallas import tpu_sc as plsc

BATCH = 16384
NF = 64
NC, NS, LANES = 2, 16, 16
NW = NC * NS
BPW = BATCH // NW
CHUNK = 128
NCH = BPW // CHUNK
GPC = CHUNK // LANES


def _nmf_body(iwr_hbm, ior_hbm, iwf_hbm, iof_hbm, w2_hbm, h2_hbm,
              wb_hbm, hb_hbm, out_hbm,
              iwr_v, ior_v, iwf_v, iof_v, bw_v, bh_v, rw_v, rh_v, o_v,
              sem0, sem1, semb):
  wid = lax.axis_index("s") * NC + lax.axis_index("c")

  pltpu.sync_copy(iwr_hbm.at[wid], iwr_v)
  pltpu.sync_copy(ior_hbm.at[wid], ior_v)
  pltpu.sync_copy(iwf_hbm.at[wid], iwf_v)
  pltpu.sync_copy(iof_hbm.at[wid], iof_v)

  for j in range(NCH):
    sl = pl.ds(j * CHUNK, CHUNK)
    pltpu.async_copy(wb_hbm.at[iwf_v.at[sl]], bw_v.at[sl], semb)
    pltpu.async_copy(hb_hbm.at[iof_v.at[sl]], bh_v.at[sl], semb)

  lanes = lax.iota(jnp.int32, LANES)

  def fire(j, slot):
    sl = pl.ds(j * CHUNK, CHUNK)
    sem = sem0 if slot == 0 else sem1
    pltpu.async_copy(w2_hbm.at[iwr_v.at[sl]], rw_v.at[slot], sem)
    pltpu.async_copy(h2_hbm.at[ior_v.at[sl]], rh_v.at[slot], sem)

  def drain(slot):
    sem = sem0 if slot == 0 else sem1
    pltpu.make_async_copy(w2_hbm.at[pl.ds(0, CHUNK), :], rw_v.at[slot], sem).wait()
    pltpu.make_async_copy(h2_hbm.at[pl.ds(0, CHUNK), :], rh_v.at[slot], sem).wait()

  def compute(j, slot):
    def group(gg, carry):
      base = j * CHUNK + gg * LANES
      nvw = iwf_v[pl.ds(base, LANES)]
      nvh = iof_v[pl.ds(base, LANES)]
      acc = jnp.zeros((LANES,), jnp.float32)
      for i in range(LANES):
        r = gg * LANES + i
        offw = (nvw[i] & 1) * NF
        offh = (nvh[i] & 1) * NF
        s = (rw_v[slot, r, pl.ds(offw, LANES)] *
             rh_v[slot, r, pl.ds(offh, LANES)])
        for k in range(1, NF // LANES):
          s = s + (rw_v[slot, r, pl.ds(offw + k * LANES, LANES)] *
                   rh_v[slot, r, pl.ds(offh + k * LANES, LANES)])
        acc = jnp.where(lanes == i, jnp.sum(s), acc)
      bsl = pl.ds(base, LANES)
      o_v[bsl] = acc + bw_v[bsl] + bh_v[bsl]
      return carry
    lax.fori_loop(0, GPC, group, 0)

  pltpu.make_async_copy(wb_hbm.at[pl.ds(0, BPW)], bw_v, semb).wait()
  pltpu.make_async_copy(hb_hbm.at[pl.ds(0, BPW)], bh_v, semb).wait()

  fire(0, 0)
  fire(1, 1)
  drain(0)
  compute(0, 0)
  fire(2, 0)
  drain(1)
  compute(1, 1)
  fire(3, 1)
  drain(0)
  compute(2, 0)
  drain(1)
  compute(3, 1)

  pltpu.sync_copy(o_v, out_hbm.at[pl.ds(wid * BPW, BPW)])


def kernel(nodes, W, H, w_bias, h_bias):
  nodes = nodes.astype(jnp.int32)
  iw = nodes[:, 0]
  io = nodes[:, 1]

  mesh = plsc.VectorSubcoreMesh(core_axis_name="c", subcore_axis_name="s",
                                num_cores=NC, num_subcores=NS)
  f = pl.kernel(
      _nmf_body,
      out_type=jax.ShapeDtypeStruct((BATCH,), jnp.float32),
      mesh=mesh,
      compiler_params=pltpu.CompilerParams(needs_layout_passes=False),
      scratch_types=[
          pltpu.VMEM((BPW,), jnp.int32),
          pltpu.VMEM((BPW,), jnp.int32),
          pltpu.VMEM((BPW,), jnp.int32),
          pltpu.VMEM((BPW,), jnp.int32),
          pltpu.VMEM((BPW,), jnp.float32),
          pltpu.VMEM((BPW,), jnp.float32),
          pltpu.VMEM((2, CHUNK, 2 * NF), jnp.float32),
          pltpu.VMEM((2, CHUNK, 2 * NF), jnp.float32),
          pltpu.VMEM((BPW,), jnp.float32),
          pltpu.SemaphoreType.DMA,
          pltpu.SemaphoreType.DMA,
          pltpu.SemaphoreType.DMA,
      ],
  )
  return f((iw >> 1).reshape(NW, BPW), (io >> 1).reshape(NW, BPW),
           iw.reshape(NW, BPW), io.reshape(NW, BPW),
           W.reshape(500000, 2 * NF), H.reshape(500000, 2 * NF),
           w_bias.reshape(-1), h_bias.reshape(-1))


# speedup vs baseline: 1.4009x; 1.4009x over previous
import functools

import jax
import jax.numpy as jnp
from jax import lax
from jax.experimental import pallas as pl
from jax.experimental.pallas import tpu as pltpu
from jax.experimental.pallas import tpu_sc as plsc

BATCH = 16384
NF = 64
NC, NS, LANES = 2, 16, 16
NW = NC * NS
BPW = BATCH // NW
CHUNK = 128
NCH = BPW // CHUNK
GRP = BPW // LANES


def _nmf_body(iw_hbm, io_hbm, w_hbm, h_hbm, wb_hbm, hb_hbm, out_hbm,
              iw_v, io_v, bw_v, bh_v, cw_v, ch_v, o_v, sem0, sem1, semb):
  wid = lax.axis_index("s") * NC + lax.axis_index("c")

  pltpu.sync_copy(iw_hbm.at[wid], iw_v)
  pltpu.sync_copy(io_hbm.at[wid], io_v)

  for j in range(NCH):
    sl = pl.ds(j * CHUNK, CHUNK)
    pltpu.async_copy(wb_hbm.at[iw_v.at[sl]], bw_v.at[sl], semb)
    pltpu.async_copy(hb_hbm.at[io_v.at[sl]], bh_v.at[sl], semb)

  lanes = lax.iota(jnp.int32, LANES)

  def fire(g, slot):
    ivw = iw_v[pl.ds(g * LANES, LANES)]
    ivh = io_v[pl.ds(g * LANES, LANES)]
    sem = sem0 if slot == 0 else sem1
    for i in range(LANES):
      pltpu.async_copy(w_hbm.at[ivw[i], :], cw_v.at[slot, i], sem)
      pltpu.async_copy(h_hbm.at[ivh[i], :], ch_v.at[slot, i], sem)

  def drain(slot):
    sem = sem0 if slot == 0 else sem1
    pltpu.make_async_copy(w_hbm.at[pl.ds(0, LANES), :], cw_v.at[slot], sem).wait()
    pltpu.make_async_copy(h_hbm.at[pl.ds(0, LANES), :], ch_v.at[slot], sem).wait()

  def compute(g, slot):
    acc = jnp.zeros((LANES,), jnp.float32)
    for i in range(LANES):
      s = cw_v[slot, i, pl.ds(0, LANES)] * ch_v[slot, i, pl.ds(0, LANES)]
      for k in range(1, NF // LANES):
        s = s + (cw_v[slot, i, pl.ds(k * LANES, LANES)] *
                 ch_v[slot, i, pl.ds(k * LANES, LANES)])
      acc = jnp.where(lanes == i, jnp.sum(s), acc)
    base = pl.ds(g * LANES, LANES)
    o_v[base] = acc + bw_v[base] + bh_v[base]

  pltpu.make_async_copy(wb_hbm.at[pl.ds(0, BPW)], bw_v, semb).wait()
  pltpu.make_async_copy(hb_hbm.at[pl.ds(0, BPW)], bh_v, semb).wait()

  fire(0, 0)
  fire(1, 1)

  def body(m, carry):
    g0 = 2 * m
    drain(0)
    compute(g0, 0)
    fire(g0 + 2, 0)
    drain(1)
    compute(g0 + 1, 1)
    fire(g0 + 3, 1)
    return carry

  lax.fori_loop(0, GRP // 2 - 1, body, 0)
  drain(0)
  compute(GRP - 2, 0)
  drain(1)
  compute(GRP - 1, 1)

  pltpu.sync_copy(o_v, out_hbm.at[pl.ds(wid * BPW, BPW)])


def kernel(nodes, W, H, w_bias, h_bias):
  nodes = nodes.astype(jnp.int32)
  iw = nodes[:, 0].reshape(NW, BPW)
  io = nodes[:, 1].reshape(NW, BPW)

  mesh = plsc.VectorSubcoreMesh(core_axis_name="c", subcore_axis_name="s",
                                num_cores=NC, num_subcores=NS)
  f = pl.kernel(
      _nmf_body,
      out_type=jax.ShapeDtypeStruct((BATCH,), jnp.float32),
      mesh=mesh,
      compiler_params=pltpu.CompilerParams(needs_layout_passes=False),
      scratch_types=[
          pltpu.VMEM((BPW,), jnp.int32),
          pltpu.VMEM((BPW,), jnp.int32),
          pltpu.VMEM((BPW,), jnp.float32),
          pltpu.VMEM((BPW,), jnp.float32),
          pltpu.VMEM((2, LANES, NF), jnp.float32),
          pltpu.VMEM((2, LANES, NF), jnp.float32),
          pltpu.VMEM((BPW,), jnp.float32),
          pltpu.SemaphoreType.DMA,
          pltpu.SemaphoreType.DMA,
          pltpu.SemaphoreType.DMA,
      ],
  )
  return f(iw, io, W, H, w_bias.reshape(-1), h_bias.reshape(-1))


# speedup vs baseline: 2.7436x; 1.9584x over previous
import functools

import jax
import jax.numpy as jnp
from jax import lax
from jax.experimental import pallas as pl
from jax.experimental.pallas import tpu as pltpu
from jax.experimental.pallas import tpu_sc as plsc

BATCH = 16384
NNODE = 1000000
NF = 64
NFP = 80
NC, NS, LANES = 2, 16, 16
NW = NC * NS
BPW = BATCH // NW
GRP = BPW // LANES
BLK = 128
LASTB = (NNODE - 1) // BLK
LASTW = NNODE - LASTB * BLK
SENT = 2 ** 30


def _scan_table(wid, bias_lane, idx_hbm, tab_hbm, bias_hbm, tail_hbm,
                btail_hbm, out_hbm,
                si_v, blk0_v, blk1_v, bb0_v, bb1_v, st_v, sem0, sem1):
  pltpu.sync_copy(idx_hbm.at[wid], si_v.at[pl.ds(0, BPW)])
  si_v[pl.ds(BPW, LANES)] = jnp.full((LANES,), SENT, jnp.int32)

  b_lo = si_v[pl.ds(0, LANES)][0] // BLK
  b_hi = si_v[pl.ds(BPW - LANES, LANES)][LANES - 1] // BLK
  nb = b_hi - b_lo + 1

  lanes = lax.iota(jnp.int32, LANES)
  rows = [lanes + k * LANES for k in range(NF // LANES)]
  one_at = jnp.where(lanes == 1 - bias_lane, 1.0, 0.0).astype(jnp.float32)

  def fire(b, blk_v, bb_v, sem):
    @pl.when(jnp.logical_and(b <= b_hi, b < LASTB))
    def _():
      start = pl.multiple_of(b * BLK, BLK)
      pltpu.async_copy(tab_hbm.at[:, pl.ds(start, BLK)], blk_v, sem)
      pltpu.async_copy(bias_hbm.at[:, pl.ds(start, BLK)], bb_v, sem)

    @pl.when(jnp.logical_and(b <= b_hi, b == LASTB))
    def _():
      pltpu.async_copy(tail_hbm, blk_v, sem)
      pltpu.async_copy(btail_hbm, bb_v, sem)

  def wait(blk_v, bb_v, sem):
    pltpu.make_async_copy(tab_hbm.at[:, pl.ds(0, BLK)], blk_v, sem).wait()
    pltpu.make_async_copy(bias_hbm.at[:, pl.ds(0, BLK)], bb_v, sem).wait()

  def serve(b, blk_v, bb_v, c0):
    start = b * BLK

    def cond(c):
      w = si_v[pl.ds(c, LANES)]
      return jnp.logical_and(c < BPW, w[0] // BLK == b)

    def body(c):
      w = si_v[pl.ds(c, LANES)]
      l = jnp.full((LANES,), w[0] - start, jnp.int32)
      for k in range(NF // LANES):
        col = plsc.load_gather(blk_v, [rows[k], l])
        st_v[c, pl.ds(k * LANES, LANES)] = col
      bv = plsc.load_gather(bb_v, [jnp.zeros((LANES,), jnp.int32), l])
      st_v[c, pl.ds(NF, LANES)] = jnp.where(lanes == bias_lane, bv, one_at)
      return c + 1

    return lax.while_loop(cond, body, c0)

  fire(b_lo, blk0_v, bb0_v, sem0)

  def biter(m, c):
    b0 = b_lo + 2 * m
    fire(b0 + 1, blk1_v, bb1_v, sem1)
    wait(blk0_v, bb0_v, sem0)
    c = serve(b0, blk0_v, bb0_v, c)
    fire(b0 + 2, blk0_v, bb0_v, sem0)

    def odd(cc):
      wait(blk1_v, bb1_v, sem1)
      return serve(b0 + 1, blk1_v, bb1_v, cc)

    return lax.cond(b0 + 1 <= b_hi, odd, lambda cc: cc, c)

  lax.fori_loop(0, (nb + 1) // 2, biter, 0)
  pltpu.sync_copy(st_v, out_hbm.at[pl.ds(wid * BPW, BPW), :])


def _scan_body(siw_hbm, sio_hbm, wt_hbm, ht_hbm, wbt_hbm, hbt_hbm,
               wtail_hbm, htail_hbm, wbtail_hbm, hbtail_hbm,
               sw_hbm, sh_hbm,
               si_v, blk0_v, blk1_v, bb0_v, bb1_v, st_v, sem0, sem1):
  wid = lax.axis_index("s") * NC + lax.axis_index("c")
  _scan_table(wid, 0, siw_hbm, wt_hbm, wbt_hbm, wtail_hbm, wbtail_hbm,
              sw_hbm, si_v, blk0_v, blk1_v, bb0_v, bb1_v, st_v, sem0, sem1)
  _scan_table(wid, 1, sio_hbm, ht_hbm, hbt_hbm, htail_hbm, hbtail_hbm,
              sh_hbm, si_v, blk0_v, blk1_v, bb0_v, bb1_v, st_v, sem0, sem1)


def _dot_body(rw_hbm, rh_hbm, sw_hbm, sh_hbm, out_hbm,
              rw_v, rh_v, cw_v, ch_v, o_v, sem0, sem1):
  wid = lax.axis_index("s") * NC + lax.axis_index("c")

  pltpu.sync_copy(rw_hbm.at[wid], rw_v)
  pltpu.sync_copy(rh_hbm.at[wid], rh_v)

  lanes = lax.iota(jnp.int32, LANES)

  def fire(g, slot):
    ivw = rw_v[pl.ds(g * LANES, LANES)]
    ivh = rh_v[pl.ds(g * LANES, LANES)]
    sem = sem0 if slot == 0 else sem1
    for i in range(LANES):
      pltpu.async_copy(sw_hbm.at[ivw[i], :], cw_v.at[slot, i], sem)
      pltpu.async_copy(sh_hbm.at[ivh[i], :], ch_v.at[slot, i], sem)

  def drain(slot):
    sem = sem0 if slot == 0 else sem1
    pltpu.make_async_copy(sw_hbm.at[pl.ds(0, LANES), :], cw_v.at[slot], sem).wait()
    pltpu.make_async_copy(sh_hbm.at[pl.ds(0, LANES), :], ch_v.at[slot], sem).wait()

  def compute(g, slot):
    acc = jnp.zeros((LANES,), jnp.float32)
    for i in range(LANES):
      s = cw_v[slot, i, pl.ds(0, LANES)] * ch_v[slot, i, pl.ds(0, LANES)]
      for k in range(1, NFP // LANES):
        s = s + (cw_v[slot, i, pl.ds(k * LANES, LANES)] *
                 ch_v[slot, i, pl.ds(k * LANES, LANES)])
      acc = jnp.where(lanes == i, jnp.sum(s), acc)
    o_v[pl.ds(g * LANES, LANES)] = acc

  fire(0, 0)
  fire(1, 1)

  def body(m, carry):
    g0 = 2 * m
    drain(0)
    compute(g0, 0)
    fire(g0 + 2, 0)
    drain(1)
    compute(g0 + 1, 1)
    fire(g0 + 3, 1)
    return carry

  lax.fori_loop(0, GRP // 2 - 1, body, 0)
  drain(0)
  compute(GRP - 2, 0)
  drain(1)
  compute(GRP - 1, 1)

  pltpu.sync_copy(o_v, out_hbm.at[pl.ds(wid * BPW, BPW)])


def kernel(nodes, W, H, w_bias, h_bias):
  nodes = nodes.astype(jnp.int32)
  iw = nodes[:, 0]
  io = nodes[:, 1]

  pos = jnp.arange(BATCH, dtype=jnp.int32)
  perm_w = jnp.argsort(iw)
  perm_h = jnp.argsort(io)
  siw = jnp.take(iw, perm_w)
  sio = jnp.take(io, perm_h)
  rank_w = jnp.zeros((BATCH,), jnp.int32).at[perm_w].set(pos)
  rank_h = jnp.zeros((BATCH,), jnp.int32).at[perm_h].set(pos)

  mesh = plsc.VectorSubcoreMesh(core_axis_name="c", subcore_axis_name="s",
                                num_cores=NC, num_subcores=NS)
  cp = pltpu.CompilerParams(needs_layout_passes=False)

  scan = pl.kernel(
      _scan_body,
      out_type=(jax.ShapeDtypeStruct((BATCH, NFP), jnp.float32),
                jax.ShapeDtypeStruct((BATCH, NFP), jnp.float32)),
      mesh=mesh,
      compiler_params=cp,
      scratch_types=[
          pltpu.VMEM((BPW + LANES,), jnp.int32),
          pltpu.VMEM((NF, BLK), jnp.float32),
          pltpu.VMEM((NF, BLK), jnp.float32),
          pltpu.VMEM((1, BLK), jnp.float32),
          pltpu.VMEM((1, BLK), jnp.float32),
          pltpu.VMEM((BPW, NFP), jnp.float32),
          pltpu.SemaphoreType.DMA,
          pltpu.SemaphoreType.DMA,
      ],
  )
  pad = ((0, 0), (0, BLK - LASTW))
  wtail = jnp.pad(W[LASTB * BLK:].T, pad)
  htail = jnp.pad(H[LASTB * BLK:].T, pad)
  wbtail = jnp.pad(w_bias[LASTB * BLK:].T, pad)
  hbtail = jnp.pad(h_bias[LASTB * BLK:].T, pad)
  sw, sh = scan(siw.reshape(NW, BPW), sio.reshape(NW, BPW),
                W.T, H.T, w_bias.T, h_bias.T,
                wtail, htail, wbtail, hbtail)

  dot = pl.kernel(
      _dot_body,
      out_type=jax.ShapeDtypeStruct((BATCH,), jnp.float32),
      mesh=mesh,
      compiler_params=cp,
      scratch_types=[
          pltpu.VMEM((BPW,), jnp.int32),
          pltpu.VMEM((BPW,), jnp.int32),
          pltpu.VMEM((2, LANES, NFP), jnp.float32),
          pltpu.VMEM((2, LANES, NFP), jnp.float32),
          pltpu.VMEM((BPW,), jnp.float32),
          pltpu.SemaphoreType.DMA,
          pltpu.SemaphoreType.DMA,
      ],
  )
  return dot(rank_w.reshape(NW, BPW), rank_h.reshape(NW, BPW), sw, sh)


# speedup vs baseline: 3.5042x; 1.2772x over previous
import functools

import jax
import jax.numpy as jnp
from jax import lax
from jax.experimental import pallas as pl
from jax.experimental.pallas import tpu as pltpu
from jax.experimental.pallas import tpu_sc as plsc

BATCH = 16384
NNODE = 1000000
NF = 64
NFP = 80
NC, NS, LANES = 2, 16, 16
NW = NC * NS
BPW = BATCH // NW
GRP = BPW // LANES
BLK = 384
LASTB = (NNODE - 1) // BLK
LASTW = NNODE - LASTB * BLK
SENT = 2 ** 30


def _scan_table(wid, bias_lane, idx_hbm, tab_hbm, bias_hbm, tail_hbm,
                btail_hbm, out_hbm,
                si_v, blk0_v, blk1_v, bb0_v, bb1_v, st_v, sem0, sem1):
  pltpu.sync_copy(idx_hbm.at[wid], si_v.at[pl.ds(0, BPW)])
  si_v[pl.ds(BPW, LANES)] = jnp.full((LANES,), SENT, jnp.int32)

  b_lo = si_v[pl.ds(0, LANES)][0] // BLK
  b_hi = si_v[pl.ds(BPW - LANES, LANES)][LANES - 1] // BLK
  nb = b_hi - b_lo + 1

  lanes = lax.iota(jnp.int32, LANES)
  rows = [lanes + k * LANES for k in range(NF // LANES)]
  one_at = jnp.where(lanes == 1 - bias_lane, 1.0, 0.0).astype(jnp.float32)

  def fire(b, blk_v, bb_v, sem):
    @pl.when(jnp.logical_and(b <= b_hi, b < LASTB))
    def _():
      start = pl.multiple_of(b * BLK, BLK)
      pltpu.async_copy(tab_hbm.at[:, pl.ds(start, BLK)], blk_v, sem)
      pltpu.async_copy(bias_hbm.at[:, pl.ds(start, BLK)], bb_v, sem)

    @pl.when(jnp.logical_and(b <= b_hi, b == LASTB))
    def _():
      pltpu.async_copy(tail_hbm, blk_v, sem)
      pltpu.async_copy(btail_hbm, bb_v, sem)

  def wait(blk_v, bb_v, sem):
    pltpu.make_async_copy(tab_hbm.at[:, pl.ds(0, BLK)], blk_v, sem).wait()
    pltpu.make_async_copy(bias_hbm.at[:, pl.ds(0, BLK)], bb_v, sem).wait()

  def serve(b, blk_v, bb_v, c0):
    start = b * BLK

    def cond(c):
      w = si_v[pl.ds(c, LANES)]
      return jnp.logical_and(c < BPW, w[0] // BLK == b)

    def body(c):
      w = si_v[pl.ds(c, LANES)]
      l = jnp.full((LANES,), w[0] - start, jnp.int32)
      for k in range(NF // LANES):
        col = plsc.load_gather(blk_v, [rows[k], l])
        st_v[c, pl.ds(k * LANES, LANES)] = col
      bv = plsc.load_gather(bb_v, [jnp.zeros((LANES,), jnp.int32), l])
      st_v[c, pl.ds(NF, LANES)] = jnp.where(lanes == bias_lane, bv, one_at)
      return c + 1

    return lax.while_loop(cond, body, c0)

  fire(b_lo, blk0_v, bb0_v, sem0)

  def biter(m, c):
    b0 = b_lo + 2 * m
    fire(b0 + 1, blk1_v, bb1_v, sem1)
    wait(blk0_v, bb0_v, sem0)
    c = serve(b0, blk0_v, bb0_v, c)
    fire(b0 + 2, blk0_v, bb0_v, sem0)

    def odd(cc):
      wait(blk1_v, bb1_v, sem1)
      return serve(b0 + 1, blk1_v, bb1_v, cc)

    return lax.cond(b0 + 1 <= b_hi, odd, lambda cc: cc, c)

  lax.fori_loop(0, (nb + 1) // 2, biter, 0)
  pltpu.sync_copy(st_v, out_hbm.at[pl.ds(wid * BPW, BPW), :])


def _scan_body(siw_hbm, sio_hbm, wt_hbm, ht_hbm, wbt_hbm, hbt_hbm,
               wtail_hbm, htail_hbm, wbtail_hbm, hbtail_hbm,
               sw_hbm, sh_hbm,
               si_v, blk0_v, blk1_v, bb0_v, bb1_v, st_v, sem0, sem1):
  wid = lax.axis_index("s") * NC + lax.axis_index("c")
  _scan_table(wid, 0, siw_hbm, wt_hbm, wbt_hbm, wtail_hbm, wbtail_hbm,
              sw_hbm, si_v, blk0_v, blk1_v, bb0_v, bb1_v, st_v, sem0, sem1)
  _scan_table(wid, 1, sio_hbm, ht_hbm, hbt_hbm, htail_hbm, hbtail_hbm,
              sh_hbm, si_v, blk0_v, blk1_v, bb0_v, bb1_v, st_v, sem0, sem1)


def _dot_body(rw_hbm, rh_hbm, sw_hbm, sh_hbm, out_hbm,
              rw_v, rh_v, cw_v, ch_v, o_v, sem0, sem1):
  wid = lax.axis_index("s") * NC + lax.axis_index("c")

  pltpu.sync_copy(rw_hbm.at[wid], rw_v)
  pltpu.sync_copy(rh_hbm.at[wid], rh_v)

  lanes = lax.iota(jnp.int32, LANES)

  def fire(g, slot):
    ivw = rw_v[pl.ds(g * LANES, LANES)]
    ivh = rh_v[pl.ds(g * LANES, LANES)]
    sem = sem0 if slot == 0 else sem1
    for i in range(LANES):
      pltpu.async_copy(sw_hbm.at[ivw[i], :], cw_v.at[slot, i], sem)
      pltpu.async_copy(sh_hbm.at[ivh[i], :], ch_v.at[slot, i], sem)

  def drain(slot):
    sem = sem0 if slot == 0 else sem1
    pltpu.make_async_copy(sw_hbm.at[pl.ds(0, LANES), :], cw_v.at[slot], sem).wait()
    pltpu.make_async_copy(sh_hbm.at[pl.ds(0, LANES), :], ch_v.at[slot], sem).wait()

  def compute(g, slot):
    acc = jnp.zeros((LANES,), jnp.float32)
    for i in range(LANES):
      s = cw_v[slot, i, pl.ds(0, LANES)] * ch_v[slot, i, pl.ds(0, LANES)]
      for k in range(1, NFP // LANES):
        s = s + (cw_v[slot, i, pl.ds(k * LANES, LANES)] *
                 ch_v[slot, i, pl.ds(k * LANES, LANES)])
      acc = jnp.where(lanes == i, jnp.sum(s), acc)
    o_v[pl.ds(g * LANES, LANES)] = acc

  fire(0, 0)
  fire(1, 1)

  def body(m, carry):
    g0 = 2 * m
    drain(0)
    compute(g0, 0)
    fire(g0 + 2, 0)
    drain(1)
    compute(g0 + 1, 1)
    fire(g0 + 3, 1)
    return carry

  lax.fori_loop(0, GRP // 2 - 1, body, 0)
  drain(0)
  compute(GRP - 2, 0)
  drain(1)
  compute(GRP - 1, 1)

  pltpu.sync_copy(o_v, out_hbm.at[pl.ds(wid * BPW, BPW)])


def kernel(nodes, W, H, w_bias, h_bias):
  nodes = nodes.astype(jnp.int32)
  iw = nodes[:, 0]
  io = nodes[:, 1]

  pos = jnp.arange(BATCH, dtype=jnp.int32)
  perm_w = jnp.argsort(iw)
  perm_h = jnp.argsort(io)
  siw = jnp.take(iw, perm_w)
  sio = jnp.take(io, perm_h)
  rank_w = jnp.zeros((BATCH,), jnp.int32).at[perm_w].set(pos)
  rank_h = jnp.zeros((BATCH,), jnp.int32).at[perm_h].set(pos)

  mesh = plsc.VectorSubcoreMesh(core_axis_name="c", subcore_axis_name="s",
                                num_cores=NC, num_subcores=NS)
  cp = pltpu.CompilerParams(needs_layout_passes=False)

  scan = pl.kernel(
      _scan_body,
      out_type=(jax.ShapeDtypeStruct((BATCH, NFP), jnp.float32),
                jax.ShapeDtypeStruct((BATCH, NFP), jnp.float32)),
      mesh=mesh,
      compiler_params=cp,
      scratch_types=[
          pltpu.VMEM((BPW + LANES,), jnp.int32),
          pltpu.VMEM((NF, BLK), jnp.float32),
          pltpu.VMEM((NF, BLK), jnp.float32),
          pltpu.VMEM((1, BLK), jnp.float32),
          pltpu.VMEM((1, BLK), jnp.float32),
          pltpu.VMEM((BPW, NFP), jnp.float32),
          pltpu.SemaphoreType.DMA,
          pltpu.SemaphoreType.DMA,
      ],
  )
  pad = ((0, 0), (0, BLK - LASTW))
  wtail = jnp.pad(W[LASTB * BLK:].T, pad)
  htail = jnp.pad(H[LASTB * BLK:].T, pad)
  wbtail = jnp.pad(w_bias[LASTB * BLK:].T, pad)
  hbtail = jnp.pad(h_bias[LASTB * BLK:].T, pad)
  sw, sh = scan(siw.reshape(NW, BPW), sio.reshape(NW, BPW),
                W.T, H.T, w_bias.T, h_bias.T,
                wtail, htail, wbtail, hbtail)

  dot = pl.kernel(
      _dot_body,
      out_type=jax.ShapeDtypeStruct((BATCH,), jnp.float32),
      mesh=mesh,
      compiler_params=cp,
      scratch_types=[
          pltpu.VMEM((BPW,), jnp.int32),
          pltpu.VMEM((BPW,), jnp.int32),
          pltpu.VMEM((2, LANES, NFP), jnp.float32),
          pltpu.VMEM((2, LANES, NFP), jnp.float32),
          pltpu.VMEM((BPW,), jnp.float32),
          pltpu.SemaphoreType.DMA,
          pltpu.SemaphoreType.DMA,
      ],
  )
  return dot(rank_w.reshape(NW, BPW), rank_h.reshape(NW, BPW), sw, sh)


# speedup vs baseline: 3.5900x; 1.0245x over previous
import functools

import jax
import jax.numpy as jnp
from jax import lax
from jax.experimental import pallas as pl
from jax.experimental.pallas import tpu as pltpu
from jax.experimental.pallas import tpu_sc as plsc

BATCH = 16384
NNODE = 1000000
NF = 64
NC, NS, LANES = 2, 16, 16
NW = NC * NS
BPW = BATCH // NW
CHUNK = 128
NCH = BPW // CHUNK
GRP = BPW // LANES
BLK = 512
LASTB = (NNODE - 1) // BLK
LASTW = NNODE - LASTB * BLK
SENT = 2 ** 30


def _scan_table(wid, idx_hbm, tab_hbm, bias_hbm, tail_hbm, btail_hbm,
                rows_hbm, sb_hbm,
                si_v, blk0_v, blk1_v, bb0_v, bb1_v, st_v, sb_v, sem0, sem1):
  pltpu.sync_copy(idx_hbm.at[wid], si_v.at[pl.ds(0, BPW)])
  si_v[pl.ds(BPW, LANES)] = jnp.full((LANES,), SENT, jnp.int32)

  b_lo = si_v[pl.ds(0, LANES)][0] // BLK
  b_hi = si_v[pl.ds(BPW - LANES, LANES)][LANES - 1] // BLK
  nb = b_hi - b_lo + 1

  lanes = lax.iota(jnp.int32, LANES)
  rows = [lanes + k * LANES for k in range(NF // LANES)]
  zeros16 = jnp.zeros((LANES,), jnp.int32)
  lane0 = lanes == 0

  def fire(b, blk_v, bb_v, sem):
    @pl.when(jnp.logical_and(b <= b_hi, b < LASTB))
    def _():
      start = pl.multiple_of(b * BLK, BLK)
      pltpu.async_copy(tab_hbm.at[:, pl.ds(start, BLK)], blk_v, sem)
      pltpu.async_copy(bias_hbm.at[:, pl.ds(start, BLK)], bb_v, sem)

    @pl.when(jnp.logical_and(b <= b_hi, b == LASTB))
    def _():
      pltpu.async_copy(tail_hbm, blk_v, sem)
      pltpu.async_copy(btail_hbm, bb_v, sem)

  def wait(blk_v, bb_v, sem):
    pltpu.make_async_copy(tab_hbm.at[:, pl.ds(0, BLK)], blk_v, sem).wait()
    pltpu.make_async_copy(bias_hbm.at[:, pl.ds(0, BLK)], bb_v, sem).wait()

  def serve(b, blk_v, bb_v, c0):
    start = b * BLK

    def cond(c):
      w = si_v[pl.ds(c, LANES)]
      return jnp.logical_and(c < BPW, w[0] // BLK == b)

    def body(c):
      w = si_v[pl.ds(c, LANES)]
      l = jnp.full((LANES,), w[0] - start, jnp.int32)
      for k in range(NF // LANES):
        col = plsc.load_gather(blk_v, [rows[k], l])
        plsc.store_scatter(st_v, [c * NF + k * LANES + lanes], col)
      bv = plsc.load_gather(bb_v, [zeros16, l])
      plsc.store_scatter(sb_v, [jnp.full((LANES,), c, jnp.int32)], bv,
                         mask=lane0)
      return c + 1

    return lax.while_loop(cond, body, c0)

  fire(b_lo, blk0_v, bb0_v, sem0)

  def biter(m, c):
    b0 = b_lo + 2 * m
    fire(b0 + 1, blk1_v, bb1_v, sem1)
    wait(blk0_v, bb0_v, sem0)
    c = serve(b0, blk0_v, bb0_v, c)
    fire(b0 + 2, blk0_v, bb0_v, sem0)

    def odd(cc):
      wait(blk1_v, bb1_v, sem1)
      return serve(b0 + 1, blk1_v, bb1_v, cc)

    return lax.cond(b0 + 1 <= b_hi, odd, lambda cc: cc, c)

  lax.fori_loop(0, (nb + 1) // 2, biter, 0)
  pltpu.sync_copy(st_v, rows_hbm.at[pl.ds(wid * BPW * NF, BPW * NF)])
  pltpu.sync_copy(sb_v, sb_hbm.at[pl.ds(wid * BPW, BPW)])


def _scan_body(siw_hbm, sio_hbm, wt_hbm, ht_hbm, wbt_hbm, hbt_hbm,
               wtail_hbm, htail_hbm, wbtail_hbm, hbtail_hbm,
               sw_hbm, sh_hbm, sbw_hbm, sbh_hbm,
               si_v, blk0_v, blk1_v, bb0_v, bb1_v, st_v, sb_v, sem0, sem1):
  wid = lax.axis_index("s") * NC + lax.axis_index("c")
  _scan_table(wid, siw_hbm, wt_hbm, wbt_hbm, wtail_hbm, wbtail_hbm,
              sw_hbm, sbw_hbm,
              si_v, blk0_v, blk1_v, bb0_v, bb1_v, st_v, sb_v, sem0, sem1)
  _scan_table(wid, sio_hbm, ht_hbm, hbt_hbm, htail_hbm, hbtail_hbm,
              sh_hbm, sbh_hbm,
              si_v, blk0_v, blk1_v, bb0_v, bb1_v, st_v, sb_v, sem0, sem1)


def _dot_body(rw_hbm, rh_hbm, sw_hbm, sh_hbm, sbw_hbm, sbh_hbm, out_hbm,
              rw_v, rh_v, bw_v, bh_v, cw_v, ch_v, o_v, sem0, sem1, semb):
  wid = lax.axis_index("s") * NC + lax.axis_index("c")

  pltpu.sync_copy(rw_hbm.at[wid], rw_v)
  pltpu.sync_copy(rh_hbm.at[wid], rh_v)

  for j in range(NCH):
    sl = pl.ds(j * CHUNK, CHUNK)
    pltpu.async_copy(sbw_hbm.at[rw_v.at[sl]], bw_v.at[sl], semb)
    pltpu.async_copy(sbh_hbm.at[rh_v.at[sl]], bh_v.at[sl], semb)

  lanes = lax.iota(jnp.int32, LANES)

  def fire(g, slot):
    ivw = rw_v[pl.ds(g * LANES, LANES)]
    ivh = rh_v[pl.ds(g * LANES, LANES)]
    sem = sem0 if slot == 0 else sem1
    for i in range(LANES):
      pltpu.async_copy(sw_hbm.at[pl.ds(ivw[i] * NF, NF)], cw_v.at[slot, i], sem)
      pltpu.async_copy(sh_hbm.at[pl.ds(ivh[i] * NF, NF)], ch_v.at[slot, i], sem)

  def drain(slot):
    sem = sem0 if slot == 0 else sem1
    for i in range(LANES):
      pltpu.make_async_copy(sw_hbm.at[pl.ds(0, NF)], cw_v.at[slot, i], sem).wait()
      pltpu.make_async_copy(sh_hbm.at[pl.ds(0, NF)], ch_v.at[slot, i], sem).wait()

  def compute(g, slot):
    acc = jnp.zeros((LANES,), jnp.float32)
    for i in range(LANES):
      s = cw_v[slot, i, pl.ds(0, LANES)] * ch_v[slot, i, pl.ds(0, LANES)]
      for k in range(1, NF // LANES):
        s = s + (cw_v[slot, i, pl.ds(k * LANES, LANES)] *
                 ch_v[slot, i, pl.ds(k * LANES, LANES)])
      acc = jnp.where(lanes == i, jnp.sum(s), acc)
    base = pl.ds(g * LANES, LANES)
    o_v[base] = acc + bw_v[base] + bh_v[base]

  pltpu.make_async_copy(sbw_hbm.at[pl.ds(0, BPW)], bw_v, semb).wait()
  pltpu.make_async_copy(sbh_hbm.at[pl.ds(0, BPW)], bh_v, semb).wait()

  fire(0, 0)
  fire(1, 1)

  def body(m, carry):
    g0 = 2 * m
    drain(0)
    compute(g0, 0)
    fire(g0 + 2, 0)
    drain(1)
    compute(g0 + 1, 1)
    fire(g0 + 3, 1)
    return carry

  lax.fori_loop(0, GRP // 2 - 1, body, 0)
  drain(0)
  compute(GRP - 2, 0)
  drain(1)
  compute(GRP - 1, 1)

  pltpu.sync_copy(o_v, out_hbm.at[pl.ds(wid * BPW, BPW)])


def kernel(nodes, W, H, w_bias, h_bias):
  nodes = nodes.astype(jnp.int32)
  iw = nodes[:, 0]
  io = nodes[:, 1]

  pos = jnp.arange(BATCH, dtype=jnp.int32)
  perm_w = jnp.argsort(iw)
  perm_h = jnp.argsort(io)
  siw = jnp.take(iw, perm_w)
  sio = jnp.take(io, perm_h)
  rank_w = jnp.zeros((BATCH,), jnp.int32).at[perm_w].set(pos)
  rank_h = jnp.zeros((BATCH,), jnp.int32).at[perm_h].set(pos)

  mesh = plsc.VectorSubcoreMesh(core_axis_name="c", subcore_axis_name="s",
                                num_cores=NC, num_subcores=NS)
  cp = pltpu.CompilerParams(needs_layout_passes=False)

  scan = pl.kernel(
      _scan_body,
      out_type=(jax.ShapeDtypeStruct((BATCH * NF,), jnp.float32),
                jax.ShapeDtypeStruct((BATCH * NF,), jnp.float32),
                jax.ShapeDtypeStruct((BATCH,), jnp.float32),
                jax.ShapeDtypeStruct((BATCH,), jnp.float32)),
      mesh=mesh,
      compiler_params=cp,
      scratch_types=[
          pltpu.VMEM((BPW + LANES,), jnp.int32),
          pltpu.VMEM((NF, BLK), jnp.float32),
          pltpu.VMEM((NF, BLK), jnp.float32),
          pltpu.VMEM((1, BLK), jnp.float32),
          pltpu.VMEM((1, BLK), jnp.float32),
          pltpu.VMEM((BPW * NF,), jnp.float32),
          pltpu.VMEM((BPW,), jnp.float32),
          pltpu.SemaphoreType.DMA,
          pltpu.SemaphoreType.DMA,
      ],
  )
  pad = ((0, 0), (0, BLK - LASTW))
  wtail = jnp.pad(W[LASTB * BLK:].T, pad)
  htail = jnp.pad(H[LASTB * BLK:].T, pad)
  wbtail = jnp.pad(w_bias[LASTB * BLK:].T, pad)
  hbtail = jnp.pad(h_bias[LASTB * BLK:].T, pad)
  sw, sh, sbw, sbh = scan(siw.reshape(NW, BPW), sio.reshape(NW, BPW),
                          W.T, H.T, w_bias.T, h_bias.T,
                          wtail, htail, wbtail, hbtail)

  dot = pl.kernel(
      _dot_body,
      out_type=jax.ShapeDtypeStruct((BATCH,), jnp.float32),
      mesh=mesh,
      compiler_params=cp,
      scratch_types=[
          pltpu.VMEM((BPW,), jnp.int32),
          pltpu.VMEM((BPW,), jnp.int32),
          pltpu.VMEM((BPW,), jnp.float32),
          pltpu.VMEM((BPW,), jnp.float32),
          pltpu.VMEM((2, LANES, NF), jnp.float32),
          pltpu.VMEM((2, LANES, NF), jnp.float32),
          pltpu.VMEM((BPW,), jnp.float32),
          pltpu.SemaphoreType.DMA,
          pltpu.SemaphoreType.DMA,
          pltpu.SemaphoreType.DMA,
      ],
  )
  return dot(rank_w.reshape(NW, BPW), rank_h.reshape(NW, BPW),
             sw, sh, sbw, sbh)


# speedup vs baseline: 3.7530x; 1.0454x over previous
import functools

import jax
import jax.numpy as jnp
from jax import lax
from jax.experimental import pallas as pl
from jax.experimental.pallas import tpu as pltpu
from jax.experimental.pallas import tpu_sc as plsc

BATCH = 16384
NNODE = 1000000
NF = 64
NC, NS, LANES = 2, 16, 16
NW = NC * NS
BPW = BATCH // NW
CHUNK = 128
NCH = BPW // CHUNK
GRP = BPW // LANES
BLK = 640
LASTB = (NNODE - 1) // BLK
LASTW = NNODE - LASTB * BLK
SENT = 2 ** 30


def _scan_table(wid, idx_hbm, tab_hbm, bias_hbm, tail_hbm, btail_hbm,
                rows_hbm, sb_hbm,
                si_v, blk0_v, blk1_v, bb0_v, bb1_v, st_v, sb_v, sem0, sem1):
  pltpu.sync_copy(idx_hbm.at[wid], si_v.at[pl.ds(0, BPW)])
  si_v[pl.ds(BPW, LANES)] = jnp.full((LANES,), SENT, jnp.int32)

  b_lo = si_v[pl.ds(0, LANES)][0] // BLK
  b_hi = si_v[pl.ds(BPW - LANES, LANES)][LANES - 1] // BLK
  nb = b_hi - b_lo + 1

  lanes = lax.iota(jnp.int32, LANES)
  rows = [lanes + k * LANES for k in range(NF // LANES)]
  zeros16 = jnp.zeros((LANES,), jnp.int32)
  lane0 = lanes == 0

  def fire(b, blk_v, bb_v, sem):
    @pl.when(jnp.logical_and(b <= b_hi, b < LASTB))
    def _():
      start = pl.multiple_of(b * BLK, BLK)
      pltpu.async_copy(tab_hbm.at[:, pl.ds(start, BLK)], blk_v, sem)
      pltpu.async_copy(bias_hbm.at[0, pl.ds(start, BLK)], bb_v, sem)

    @pl.when(jnp.logical_and(b <= b_hi, b == LASTB))
    def _():
      pltpu.async_copy(tail_hbm, blk_v, sem)
      pltpu.async_copy(btail_hbm, bb_v, sem)

  def wait(blk_v, bb_v, sem):
    pltpu.make_async_copy(tab_hbm.at[:, pl.ds(0, BLK)], blk_v, sem).wait()
    pltpu.make_async_copy(bias_hbm.at[0, pl.ds(0, BLK)], bb_v, sem).wait()

  def serve(b, blk_v, bb_v, c0):
    start = b * BLK

    def cond(c):
      w = si_v[pl.ds(c, LANES)]
      return jnp.logical_and(c < BPW, w[0] // BLK == b)

    def body(c):
      w = si_v[pl.ds(c, LANES)]
      l = jnp.full((LANES,), w[0] - start, jnp.int32)
      for k in range(NF // LANES):
        col = plsc.load_gather(blk_v, [rows[k], l])
        plsc.store_scatter(st_v, [c * NF + k * LANES + lanes], col)
      bv = plsc.load_gather(bb_v, [l])
      plsc.store_scatter(sb_v, [jnp.full((LANES,), c, jnp.int32)], bv,
                         mask=lane0)
      return c + 1

    return lax.while_loop(cond, body, c0)

  fire(b_lo, blk0_v, bb0_v, sem0)

  def biter(m, c):
    b0 = b_lo + 2 * m
    fire(b0 + 1, blk1_v, bb1_v, sem1)
    wait(blk0_v, bb0_v, sem0)
    c = serve(b0, blk0_v, bb0_v, c)
    fire(b0 + 2, blk0_v, bb0_v, sem0)

    def odd(cc):
      wait(blk1_v, bb1_v, sem1)
      return serve(b0 + 1, blk1_v, bb1_v, cc)

    return lax.cond(b0 + 1 <= b_hi, odd, lambda cc: cc, c)

  lax.fori_loop(0, (nb + 1) // 2, biter, 0)
  pltpu.sync_copy(st_v, rows_hbm.at[pl.ds(wid * BPW * NF, BPW * NF)])
  pltpu.sync_copy(sb_v, sb_hbm.at[pl.ds(wid * BPW, BPW)])


def _scan_body(siw_hbm, sio_hbm, wt_hbm, ht_hbm, wbt_hbm, hbt_hbm,
               wtail_hbm, htail_hbm, wbtail_hbm, hbtail_hbm,
               sw_hbm, sh_hbm, sbw_hbm, sbh_hbm,
               si_v, blk0_v, blk1_v, bb0_v, bb1_v, st_v, sb_v, sem0, sem1):
  wid = lax.axis_index("s") * NC + lax.axis_index("c")
  _scan_table(wid, siw_hbm, wt_hbm, wbt_hbm, wtail_hbm, wbtail_hbm,
              sw_hbm, sbw_hbm,
              si_v, blk0_v, blk1_v, bb0_v, bb1_v, st_v, sb_v, sem0, sem1)
  _scan_table(wid, sio_hbm, ht_hbm, hbt_hbm, htail_hbm, hbtail_hbm,
              sh_hbm, sbh_hbm,
              si_v, blk0_v, blk1_v, bb0_v, bb1_v, st_v, sb_v, sem0, sem1)


def _dot_body(rw_hbm, rh_hbm, sw_hbm, sh_hbm, sbw_hbm, sbh_hbm, out_hbm,
              rw_v, rh_v, bw_v, bh_v, cw_v, ch_v, o_v, sem0, sem1, semb):
  wid = lax.axis_index("s") * NC + lax.axis_index("c")

  pltpu.sync_copy(rw_hbm.at[wid], rw_v)
  pltpu.sync_copy(rh_hbm.at[wid], rh_v)

  for j in range(NCH):
    sl = pl.ds(j * CHUNK, CHUNK)
    pltpu.async_copy(sbw_hbm.at[rw_v.at[sl]], bw_v.at[sl], semb)
    pltpu.async_copy(sbh_hbm.at[rh_v.at[sl]], bh_v.at[sl], semb)

  lanes = lax.iota(jnp.int32, LANES)

  def fire(g, slot):
    ivw = rw_v[pl.ds(g * LANES, LANES)]
    ivh = rh_v[pl.ds(g * LANES, LANES)]
    sem = sem0 if slot == 0 else sem1
    for i in range(LANES):
      pltpu.async_copy(sw_hbm.at[pl.ds(ivw[i] * NF, NF)], cw_v.at[slot, i], sem)
      pltpu.async_copy(sh_hbm.at[pl.ds(ivh[i] * NF, NF)], ch_v.at[slot, i], sem)

  def drain(slot):
    sem = sem0 if slot == 0 else sem1
    for i in range(LANES):
      pltpu.make_async_copy(sw_hbm.at[pl.ds(0, NF)], cw_v.at[slot, i], sem).wait()
      pltpu.make_async_copy(sh_hbm.at[pl.ds(0, NF)], ch_v.at[slot, i], sem).wait()

  def compute(g, slot):
    acc = jnp.zeros((LANES,), jnp.float32)
    for i in range(LANES):
      s = cw_v[slot, i, pl.ds(0, LANES)] * ch_v[slot, i, pl.ds(0, LANES)]
      for k in range(1, NF // LANES):
        s = s + (cw_v[slot, i, pl.ds(k * LANES, LANES)] *
                 ch_v[slot, i, pl.ds(k * LANES, LANES)])
      acc = jnp.where(lanes == i, jnp.sum(s), acc)
    base = pl.ds(g * LANES, LANES)
    o_v[base] = acc + bw_v[base] + bh_v[base]

  pltpu.make_async_copy(sbw_hbm.at[pl.ds(0, BPW)], bw_v, semb).wait()
  pltpu.make_async_copy(sbh_hbm.at[pl.ds(0, BPW)], bh_v, semb).wait()

  fire(0, 0)
  fire(1, 1)

  def body(m, carry):
    g0 = 2 * m
    drain(0)
    compute(g0, 0)
    fire(g0 + 2, 0)
    drain(1)
    compute(g0 + 1, 1)
    fire(g0 + 3, 1)
    return carry

  lax.fori_loop(0, GRP // 2 - 1, body, 0)
  drain(0)
  compute(GRP - 2, 0)
  drain(1)
  compute(GRP - 1, 1)

  pltpu.sync_copy(o_v, out_hbm.at[pl.ds(wid * BPW, BPW)])


def kernel(nodes, W, H, w_bias, h_bias):
  nodes = nodes.astype(jnp.int32)
  iw = nodes[:, 0]
  io = nodes[:, 1]

  pos = jnp.arange(BATCH, dtype=jnp.int32)
  siw = jnp.sort(iw)
  sio = jnp.sort(io)
  rank_w = jnp.zeros((BATCH,), jnp.int32).at[jnp.argsort(iw)].set(pos)
  rank_h = jnp.zeros((BATCH,), jnp.int32).at[jnp.argsort(io)].set(pos)

  mesh = plsc.VectorSubcoreMesh(core_axis_name="c", subcore_axis_name="s",
                                num_cores=NC, num_subcores=NS)
  cp = pltpu.CompilerParams(needs_layout_passes=False)

  scan = pl.kernel(
      _scan_body,
      out_type=(jax.ShapeDtypeStruct((BATCH * NF,), jnp.float32),
                jax.ShapeDtypeStruct((BATCH * NF,), jnp.float32),
                jax.ShapeDtypeStruct((BATCH,), jnp.float32),
                jax.ShapeDtypeStruct((BATCH,), jnp.float32)),
      mesh=mesh,
      compiler_params=cp,
      scratch_types=[
          pltpu.VMEM((BPW + LANES,), jnp.int32),
          pltpu.VMEM((NF, BLK), jnp.float32),
          pltpu.VMEM((NF, BLK), jnp.float32),
          pltpu.VMEM((BLK,), jnp.float32),
          pltpu.VMEM((BLK,), jnp.float32),
          pltpu.VMEM((BPW * NF,), jnp.float32),
          pltpu.VMEM((BPW,), jnp.float32),
          pltpu.SemaphoreType.DMA,
          pltpu.SemaphoreType.DMA,
      ],
  )
  pad = ((0, 0), (0, BLK - LASTW))
  wtail = jnp.pad(W[LASTB * BLK:].T, pad)
  htail = jnp.pad(H[LASTB * BLK:].T, pad)
  wbtail = jnp.pad(w_bias[LASTB * BLK:, 0], (0, BLK - LASTW))
  hbtail = jnp.pad(h_bias[LASTB * BLK:, 0], (0, BLK - LASTW))
  sw, sh, sbw, sbh = scan(siw.reshape(NW, BPW), sio.reshape(NW, BPW),
                          W.T, H.T, w_bias.T, h_bias.T,
                          wtail, htail, wbtail, hbtail)

  dot = pl.kernel(
      _dot_body,
      out_type=jax.ShapeDtypeStruct((BATCH,), jnp.float32),
      mesh=mesh,
      compiler_params=cp,
      scratch_types=[
          pltpu.VMEM((BPW,), jnp.int32),
          pltpu.VMEM((BPW,), jnp.int32),
          pltpu.VMEM((BPW,), jnp.float32),
          pltpu.VMEM((BPW,), jnp.float32),
          pltpu.VMEM((2, LANES, NF), jnp.float32),
          pltpu.VMEM((2, LANES, NF), jnp.float32),
          pltpu.VMEM((BPW,), jnp.float32),
          pltpu.SemaphoreType.DMA,
          pltpu.SemaphoreType.DMA,
          pltpu.SemaphoreType.DMA,
      ],
  )
  return dot(rank_w.reshape(NW, BPW), rank_h.reshape(NW, BPW),
             sw, sh, sbw, sbh)


# speedup vs baseline: 4.1108x; 1.0953x over previous
import jax
import jax.numpy as jnp
from jax import lax
from jax.experimental import pallas as pl
from jax.experimental.pallas import tpu as pltpu
from jax.experimental.pallas import tpu_sc as plsc

BATCH = 16384
NNODE = 1000000
NF = 64
NC, NS, LANES = 2, 16, 16
NW = NC * NS
BPW = BATCH // NW
CHUNK = 128
NCH = BPW // CHUNK
GRP = BPW // LANES
BLK = 384
LASTB = (NNODE - 1) // BLK
LASTW = NNODE - LASTB * BLK
SENT = 2 ** 30


def _scan_table(wid, idx_hbm, tab_hbm, bias_hbm, tail_hbm, btail_hbm,
                rows_hbm, sb_hbm, si_v, blk0_v, blk1_v, blk2_v,
                bb0_v, bb1_v, bb2_v, st_v, sb_v, sem0, sem1, sem2):
  pltpu.sync_copy(idx_hbm.at[wid], si_v.at[pl.ds(0, BPW)])
  si_v[pl.ds(BPW, LANES)] = jnp.full((LANES,), SENT, jnp.int32)

  b_lo = si_v[pl.ds(0, LANES)][0] // BLK
  b_hi = si_v[pl.ds(BPW - LANES, LANES)][LANES - 1] // BLK
  nb = b_hi - b_lo + 1

  lanes = lax.iota(jnp.int32, LANES)
  rows = [lanes + k * LANES for k in range(NF // LANES)]
  lane0 = lanes == 0

  def fire(b, blk_v, bb_v, sem):
    @pl.when(jnp.logical_and(b <= b_hi, b < LASTB))
    def _():
      start = pl.multiple_of(b * BLK, BLK)
      pltpu.async_copy(tab_hbm.at[:, pl.ds(start, BLK)], blk_v, sem)
      pltpu.async_copy(bias_hbm.at[0, pl.ds(start, BLK)], bb_v, sem)

    @pl.when(jnp.logical_and(b <= b_hi, b == LASTB))
    def _():
      pltpu.async_copy(tail_hbm, blk_v, sem)
      pltpu.async_copy(btail_hbm, bb_v, sem)

  def wait(blk_v, bb_v, sem):
    pltpu.make_async_copy(tab_hbm.at[:, pl.ds(0, BLK)], blk_v, sem).wait()
    pltpu.make_async_copy(bias_hbm.at[0, pl.ds(0, BLK)], bb_v, sem).wait()

  def serve(b, blk_v, bb_v, c0):
    start = b * BLK

    def cond(c):
      w = si_v[pl.ds(c, LANES)]
      return jnp.logical_and(c < BPW, w[0] // BLK == b)

    def body(c):
      w = si_v[pl.ds(c, LANES)]
      l = jnp.full((LANES,), w[0] - start, jnp.int32)
      for k in range(NF // LANES):
        col = plsc.load_gather(blk_v, [rows[k], l])
        plsc.store_scatter(st_v, [c * NF + k * LANES + lanes], col)
      bv = plsc.load_gather(bb_v, [l])
      plsc.store_scatter(sb_v, [jnp.full((LANES,), c, jnp.int32)], bv,
                         mask=lane0)
      return c + 1

    return lax.while_loop(cond, body, c0)

  fire(b_lo, blk0_v, bb0_v, sem0)
  fire(b_lo + 1, blk1_v, bb1_v, sem1)
  fire(b_lo + 2, blk2_v, bb2_v, sem2)

  def step(b, blk_v, bb_v, sem, cc):
    def go(c):
      wait(blk_v, bb_v, sem)
      c = serve(b, blk_v, bb_v, c)
      fire(b + 3, blk_v, bb_v, sem)
      return c
    return lax.cond(b <= b_hi, go, lambda c: c, cc)

  def biter(m, c):
    b0 = b_lo + 3 * m
    c = step(b0, blk0_v, bb0_v, sem0, c)
    c = step(b0 + 1, blk1_v, bb1_v, sem1, c)
    c = step(b0 + 2, blk2_v, bb2_v, sem2, c)
    return c

  lax.fori_loop(0, (nb + 2) // 3, biter, 0)
  pltpu.sync_copy(st_v, rows_hbm.at[pl.ds(wid * BPW * NF, BPW * NF)])
  pltpu.sync_copy(sb_v, sb_hbm.at[pl.ds(wid * BPW, BPW)])


def _scan_body(siw_hbm, sio_hbm, wt_hbm, ht_hbm, wbt_hbm, hbt_hbm,
               wtail_hbm, htail_hbm, wbtail_hbm, hbtail_hbm,
               sw_hbm, sh_hbm, sbw_hbm, sbh_hbm,
               si_v, blk0_v, blk1_v, blk2_v, bb0_v, bb1_v, bb2_v,
               st_v, sb_v, sem0, sem1, sem2):
  wid = lax.axis_index("s") * NC + lax.axis_index("c")
  _scan_table(wid, siw_hbm, wt_hbm, wbt_hbm, wtail_hbm, wbtail_hbm,
              sw_hbm, sbw_hbm, si_v, blk0_v, blk1_v, blk2_v,
              bb0_v, bb1_v, bb2_v, st_v, sb_v, sem0, sem1, sem2)
  _scan_table(wid, sio_hbm, ht_hbm, hbt_hbm, htail_hbm, hbtail_hbm,
              sh_hbm, sbh_hbm, si_v, blk0_v, blk1_v, blk2_v,
              bb0_v, bb1_v, bb2_v, st_v, sb_v, sem0, sem1, sem2)


def _dot_body(rw_hbm, rh_hbm, sw_hbm, sh_hbm, sbw_hbm, sbh_hbm, out_hbm,
              rw_v, rh_v, bw_v, bh_v, cw_v, ch_v, o_v, sem0, sem1, semb):
  wid = lax.axis_index("s") * NC + lax.axis_index("c")

  pltpu.sync_copy(rw_hbm.at[wid], rw_v)
  pltpu.sync_copy(rh_hbm.at[wid], rh_v)

  for j in range(NCH):
    sl = pl.ds(j * CHUNK, CHUNK)
    pltpu.async_copy(sbw_hbm.at[rw_v.at[sl]], bw_v.at[sl], semb)
    pltpu.async_copy(sbh_hbm.at[rh_v.at[sl]], bh_v.at[sl], semb)

  lanes = lax.iota(jnp.int32, LANES)

  def fire(g, slot):
    ivw = rw_v[pl.ds(g * LANES, LANES)]
    ivh = rh_v[pl.ds(g * LANES, LANES)]
    sem = sem0 if slot == 0 else sem1
    for i in range(LANES):
      pltpu.async_copy(sw_hbm.at[pl.ds(ivw[i] * NF, NF)], cw_v.at[slot, i], sem)
      pltpu.async_copy(sh_hbm.at[pl.ds(ivh[i] * NF, NF)], ch_v.at[slot, i], sem)

  def drain(slot):
    sem = sem0 if slot == 0 else sem1
    for i in range(LANES):
      pltpu.make_async_copy(sw_hbm.at[pl.ds(0, NF)], cw_v.at[slot, i], sem).wait()
      pltpu.make_async_copy(sh_hbm.at[pl.ds(0, NF)], ch_v.at[slot, i], sem).wait()

  def compute(g, slot):
    acc = jnp.zeros((LANES,), jnp.float32)
    for i in range(LANES):
      s = cw_v[slot, i, pl.ds(0, LANES)] * ch_v[slot, i, pl.ds(0, LANES)]
      for k in range(1, NF // LANES):
        s = s + (cw_v[slot, i, pl.ds(k * LANES, LANES)] *
                 ch_v[slot, i, pl.ds(k * LANES, LANES)])
      acc = jnp.where(lanes == i, jnp.sum(s), acc)
    base = pl.ds(g * LANES, LANES)
    o_v[base] = acc + bw_v[base] + bh_v[base]

  pltpu.make_async_copy(sbw_hbm.at[pl.ds(0, BPW)], bw_v, semb).wait()
  pltpu.make_async_copy(sbh_hbm.at[pl.ds(0, BPW)], bh_v, semb).wait()

  fire(0, 0)
  fire(1, 1)

  def body(m, carry):
    g0 = 2 * m
    drain(0)
    compute(g0, 0)
    fire(g0 + 2, 0)
    drain(1)
    compute(g0 + 1, 1)
    fire(g0 + 3, 1)
    return carry

  lax.fori_loop(0, GRP // 2 - 1, body, 0)
  drain(0)
  compute(GRP - 2, 0)
  drain(1)
  compute(GRP - 1, 1)

  pltpu.sync_copy(o_v, out_hbm.at[pl.ds(wid * BPW, BPW)])


def kernel(nodes, W, H, w_bias, h_bias):
  nodes = nodes.astype(jnp.int32)
  iw = nodes[:, 0]
  io = nodes[:, 1]

  pos = jnp.arange(BATCH, dtype=jnp.int32)
  siw = jnp.sort(iw)
  sio = jnp.sort(io)
  rank_w = jnp.zeros((BATCH,), jnp.int32).at[jnp.argsort(iw)].set(pos)
  rank_h = jnp.zeros((BATCH,), jnp.int32).at[jnp.argsort(io)].set(pos)

  mesh = plsc.VectorSubcoreMesh(core_axis_name="c", subcore_axis_name="s",
                                num_cores=NC, num_subcores=NS)
  cp = pltpu.CompilerParams(needs_layout_passes=False)

  scan = pl.kernel(
      _scan_body,
      out_type=(jax.ShapeDtypeStruct((BATCH * NF,), jnp.float32),
                jax.ShapeDtypeStruct((BATCH * NF,), jnp.float32),
                jax.ShapeDtypeStruct((BATCH,), jnp.float32),
                jax.ShapeDtypeStruct((BATCH,), jnp.float32)),
      mesh=mesh,
      compiler_params=cp,
      scratch_types=[
          pltpu.VMEM((BPW + LANES,), jnp.int32),
          pltpu.VMEM((NF, BLK), jnp.float32),
          pltpu.VMEM((NF, BLK), jnp.float32),
          pltpu.VMEM((NF, BLK), jnp.float32),
          pltpu.VMEM((BLK,), jnp.float32),
          pltpu.VMEM((BLK,), jnp.float32),
          pltpu.VMEM((BLK,), jnp.float32),
          pltpu.VMEM((BPW * NF,), jnp.float32),
          pltpu.VMEM((BPW,), jnp.float32),
          pltpu.SemaphoreType.DMA,
          pltpu.SemaphoreType.DMA,
          pltpu.SemaphoreType.DMA,
      ],
  )
  pad = ((0, 0), (0, BLK - LASTW))
  wtail = jnp.pad(W[LASTB * BLK:].T, pad)
  htail = jnp.pad(H[LASTB * BLK:].T, pad)
  wbtail = jnp.pad(w_bias[LASTB * BLK:, 0], (0, BLK - LASTW))
  hbtail = jnp.pad(h_bias[LASTB * BLK:, 0], (0, BLK - LASTW))
  sw, sh, sbw, sbh = scan(siw.reshape(NW, BPW), sio.reshape(NW, BPW),
                          W.T, H.T, w_bias.T, h_bias.T,
                          wtail, htail, wbtail, hbtail)

  dot = pl.kernel(
      _dot_body,
      out_type=jax.ShapeDtypeStruct((BATCH,), jnp.float32),
      mesh=mesh,
      compiler_params=cp,
      scratch_types=[
          pltpu.VMEM((BPW,), jnp.int32),
          pltpu.VMEM((BPW,), jnp.int32),
          pltpu.VMEM((BPW,), jnp.float32),
          pltpu.VMEM((BPW,), jnp.float32),
          pltpu.VMEM((2, LANES, NF), jnp.float32),
          pltpu.VMEM((2, LANES, NF), jnp.float32),
          pltpu.VMEM((BPW,), jnp.float32),
          pltpu.SemaphoreType.DMA,
          pltpu.SemaphoreType.DMA,
          pltpu.SemaphoreType.DMA,
      ],
  )
  return dot(rank_w.reshape(NW, BPW), rank_h.reshape(NW, BPW),
             sw, sh, sbw, sbh)


# speedup vs baseline: 4.2557x; 1.0352x over previous
import jax
import jax.numpy as jnp
from jax import lax
from jax.experimental import pallas as pl
from jax.experimental.pallas import tpu as pltpu
from jax.experimental.pallas import tpu_sc as plsc

BATCH = 16384
NNODE = 1000000
NF = 64
NC, NS, LANES = 2, 16, 16
NW = NC * NS
BPW = BATCH // NW
CHUNK = 128
NCH = BPW // CHUNK
GRP = BPW // LANES
BLK = 256
LASTB = (NNODE - 1) // BLK
LASTW = NNODE - LASTB * BLK
SENT = 2 ** 30


def _scan_table(wid, idx_hbm, tab_hbm, bias_hbm, tail_hbm, btail_hbm,
                rows_hbm, sb_hbm, si_v, blk0_v, blk1_v, blk2_v, blk3_v,
                bb0_v, bb1_v, bb2_v, bb3_v, st_v, sb_v,
                sem0, sem1, sem2, sem3):
  pltpu.sync_copy(idx_hbm.at[wid], si_v.at[pl.ds(0, BPW)])
  si_v[pl.ds(BPW, LANES)] = jnp.full((LANES,), SENT, jnp.int32)

  b_lo = si_v[pl.ds(0, LANES)][0] // BLK
  b_hi = si_v[pl.ds(BPW - LANES, LANES)][LANES - 1] // BLK
  nb = b_hi - b_lo + 1

  lanes = lax.iota(jnp.int32, LANES)
  rows = [lanes + k * LANES for k in range(NF // LANES)]
  lane0 = lanes == 0

  def fire(b, blk_v, bb_v, sem):
    @pl.when(jnp.logical_and(b <= b_hi, b < LASTB))
    def _():
      start = pl.multiple_of(b * BLK, BLK)
      pltpu.async_copy(tab_hbm.at[:, pl.ds(start, BLK)], blk_v, sem)
      pltpu.async_copy(bias_hbm.at[0, pl.ds(start, BLK)], bb_v, sem)

    @pl.when(jnp.logical_and(b <= b_hi, b == LASTB))
    def _():
      pltpu.async_copy(tail_hbm, blk_v, sem)
      pltpu.async_copy(btail_hbm, bb_v, sem)

  def wait(blk_v, bb_v, sem):
    pltpu.make_async_copy(tab_hbm.at[:, pl.ds(0, BLK)], blk_v, sem).wait()
    pltpu.make_async_copy(bias_hbm.at[0, pl.ds(0, BLK)], bb_v, sem).wait()

  def serve(b, blk_v, bb_v, c0):
    start = b * BLK

    def cond(c):
      w = si_v[pl.ds(c, LANES)]
      return jnp.logical_and(c < BPW, w[0] // BLK == b)

    def body(c):
      w = si_v[pl.ds(c, LANES)]
      l = jnp.full((LANES,), w[0] - start, jnp.int32)
      for k in range(NF // LANES):
        col = plsc.load_gather(blk_v, [rows[k], l])
        plsc.store_scatter(st_v, [c * NF + k * LANES + lanes], col)
      bv = plsc.load_gather(bb_v, [l])
      plsc.store_scatter(sb_v, [jnp.full((LANES,), c, jnp.int32)], bv,
                         mask=lane0)
      return c + 1

    return lax.while_loop(cond, body, c0)

  fire(b_lo, blk0_v, bb0_v, sem0)
  fire(b_lo + 1, blk1_v, bb1_v, sem1)
  fire(b_lo + 2, blk2_v, bb2_v, sem2)
  fire(b_lo + 3, blk3_v, bb3_v, sem3)

  def step(b, blk_v, bb_v, sem, cc):
    def go(c):
      wait(blk_v, bb_v, sem)
      c = serve(b, blk_v, bb_v, c)
      fire(b + 4, blk_v, bb_v, sem)
      return c
    return lax.cond(b <= b_hi, go, lambda c: c, cc)

  def biter(m, c):
    b0 = b_lo + 4 * m
    c = step(b0, blk0_v, bb0_v, sem0, c)
    c = step(b0 + 1, blk1_v, bb1_v, sem1, c)
    c = step(b0 + 2, blk2_v, bb2_v, sem2, c)
    c = step(b0 + 3, blk3_v, bb3_v, sem3, c)
    return c

  lax.fori_loop(0, (nb + 3) // 4, biter, 0)
  pltpu.sync_copy(st_v, rows_hbm.at[pl.ds(wid * BPW * NF, BPW * NF)])
  pltpu.sync_copy(sb_v, sb_hbm.at[pl.ds(wid * BPW, BPW)])


def _scan_body(siw_hbm, sio_hbm, wt_hbm, ht_hbm, wbt_hbm, hbt_hbm,
               wtail_hbm, htail_hbm, wbtail_hbm, hbtail_hbm,
               sw_hbm, sh_hbm, sbw_hbm, sbh_hbm,
               si_v, blk0_v, blk1_v, blk2_v, blk3_v,
               bb0_v, bb1_v, bb2_v, bb3_v, st_v, sb_v,
               sem0, sem1, sem2, sem3):
  wid = lax.axis_index("s") * NC + lax.axis_index("c")
  _scan_table(wid, siw_hbm, wt_hbm, wbt_hbm, wtail_hbm, wbtail_hbm,
              sw_hbm, sbw_hbm, si_v, blk0_v, blk1_v, blk2_v, blk3_v,
              bb0_v, bb1_v, bb2_v, bb3_v, st_v, sb_v, sem0, sem1, sem2, sem3)
  _scan_table(wid, sio_hbm, ht_hbm, hbt_hbm, htail_hbm, hbtail_hbm,
              sh_hbm, sbh_hbm, si_v, blk0_v, blk1_v, blk2_v, blk3_v,
              bb0_v, bb1_v, bb2_v, bb3_v, st_v, sb_v, sem0, sem1, sem2, sem3)


def _dot_body(rw_hbm, rh_hbm, sw_hbm, sh_hbm, sbw_hbm, sbh_hbm, out_hbm,
              rw_v, rh_v, bw_v, bh_v, cw_v, ch_v, o_v, sem0, sem1, semb):
  wid = lax.axis_index("s") * NC + lax.axis_index("c")

  pltpu.sync_copy(rw_hbm.at[wid], rw_v)
  pltpu.sync_copy(rh_hbm.at[wid], rh_v)

  for j in range(NCH):
    sl = pl.ds(j * CHUNK, CHUNK)
    pltpu.async_copy(sbw_hbm.at[rw_v.at[sl]], bw_v.at[sl], semb)
    pltpu.async_copy(sbh_hbm.at[rh_v.at[sl]], bh_v.at[sl], semb)

  lanes = lax.iota(jnp.int32, LANES)

  def fire(g, slot):
    ivw = rw_v[pl.ds(g * LANES, LANES)]
    ivh = rh_v[pl.ds(g * LANES, LANES)]
    sem = sem0 if slot == 0 else sem1
    for i in range(LANES):
      pltpu.async_copy(sw_hbm.at[pl.ds(ivw[i] * NF, NF)], cw_v.at[slot, i], sem)
      pltpu.async_copy(sh_hbm.at[pl.ds(ivh[i] * NF, NF)], ch_v.at[slot, i], sem)

  def drain(slot):
    sem = sem0 if slot == 0 else sem1
    for i in range(LANES):
      pltpu.make_async_copy(sw_hbm.at[pl.ds(0, NF)], cw_v.at[slot, i], sem).wait()
      pltpu.make_async_copy(sh_hbm.at[pl.ds(0, NF)], ch_v.at[slot, i], sem).wait()

  def compute(g, slot):
    acc = jnp.zeros((LANES,), jnp.float32)
    for i in range(LANES):
      s = cw_v[slot, i, pl.ds(0, LANES)] * ch_v[slot, i, pl.ds(0, LANES)]
      for k in range(1, NF // LANES):
        s = s + (cw_v[slot, i, pl.ds(k * LANES, LANES)] *
                 ch_v[slot, i, pl.ds(k * LANES, LANES)])
      acc = jnp.where(lanes == i, jnp.sum(s), acc)
    base = pl.ds(g * LANES, LANES)
    o_v[base] = acc + bw_v[base] + bh_v[base]

  pltpu.make_async_copy(sbw_hbm.at[pl.ds(0, BPW)], bw_v, semb).wait()
  pltpu.make_async_copy(sbh_hbm.at[pl.ds(0, BPW)], bh_v, semb).wait()

  fire(0, 0)
  fire(1, 1)

  def body(m, carry):
    g0 = 2 * m
    drain(0)
    compute(g0, 0)
    fire(g0 + 2, 0)
    drain(1)
    compute(g0 + 1, 1)
    fire(g0 + 3, 1)
    return carry

  lax.fori_loop(0, GRP // 2 - 1, body, 0)
  drain(0)
  compute(GRP - 2, 0)
  drain(1)
  compute(GRP - 1, 1)

  pltpu.sync_copy(o_v, out_hbm.at[pl.ds(wid * BPW, BPW)])


def kernel(nodes, W, H, w_bias, h_bias):
  nodes = nodes.astype(jnp.int32)
  iw = nodes[:, 0]
  io = nodes[:, 1]

  pos = jnp.arange(BATCH, dtype=jnp.int32)
  siw = jnp.sort(iw)
  sio = jnp.sort(io)
  rank_w = jnp.zeros((BATCH,), jnp.int32).at[jnp.argsort(iw)].set(pos)
  rank_h = jnp.zeros((BATCH,), jnp.int32).at[jnp.argsort(io)].set(pos)

  mesh = plsc.VectorSubcoreMesh(core_axis_name="c", subcore_axis_name="s",
                                num_cores=NC, num_subcores=NS)
  cp = pltpu.CompilerParams(needs_layout_passes=False)

  scan = pl.kernel(
      _scan_body,
      out_type=(jax.ShapeDtypeStruct((BATCH * NF,), jnp.float32),
                jax.ShapeDtypeStruct((BATCH * NF,), jnp.float32),
                jax.ShapeDtypeStruct((BATCH,), jnp.float32),
                jax.ShapeDtypeStruct((BATCH,), jnp.float32)),
      mesh=mesh,
      compiler_params=cp,
      scratch_types=[
          pltpu.VMEM((BPW + LANES,), jnp.int32),
          pltpu.VMEM((NF, BLK), jnp.float32),
          pltpu.VMEM((NF, BLK), jnp.float32),
          pltpu.VMEM((NF, BLK), jnp.float32),
          pltpu.VMEM((NF, BLK), jnp.float32),
          pltpu.VMEM((BLK,), jnp.float32),
          pltpu.VMEM((BLK,), jnp.float32),
          pltpu.VMEM((BLK,), jnp.float32),
          pltpu.VMEM((BLK,), jnp.float32),
          pltpu.VMEM((BPW * NF,), jnp.float32),
          pltpu.VMEM((BPW,), jnp.float32),
          pltpu.SemaphoreType.DMA,
          pltpu.SemaphoreType.DMA,
          pltpu.SemaphoreType.DMA,
          pltpu.SemaphoreType.DMA,
      ],
  )
  pad = ((0, 0), (0, BLK - LASTW))
  wtail = jnp.pad(W[LASTB * BLK:].T, pad)
  htail = jnp.pad(H[LASTB * BLK:].T, pad)
  wbtail = jnp.pad(w_bias[LASTB * BLK:, 0], (0, BLK - LASTW))
  hbtail = jnp.pad(h_bias[LASTB * BLK:, 0], (0, BLK - LASTW))
  sw, sh, sbw, sbh = scan(siw.reshape(NW, BPW), sio.reshape(NW, BPW),
                          W.T, H.T, w_bias.T, h_bias.T,
                          wtail, htail, wbtail, hbtail)

  dot = pl.kernel(
      _dot_body,
      out_type=jax.ShapeDtypeStruct((BATCH,), jnp.float32),
      mesh=mesh,
      compiler_params=cp,
      scratch_types=[
          pltpu.VMEM((BPW,), jnp.int32),
          pltpu.VMEM((BPW,), jnp.int32),
          pltpu.VMEM((BPW,), jnp.float32),
          pltpu.VMEM((BPW,), jnp.float32),
          pltpu.VMEM((2, LANES, NF), jnp.float32),
          pltpu.VMEM((2, LANES, NF), jnp.float32),
          pltpu.VMEM((BPW,), jnp.float32),
          pltpu.SemaphoreType.DMA,
          pltpu.SemaphoreType.DMA,
          pltpu.SemaphoreType.DMA,
      ],
  )
  return dot(rank_w.reshape(NW, BPW), rank_h.reshape(NW, BPW),
             sw, sh, sbw, sbh)


# speedup vs baseline: 4.3747x; 1.0280x over previous
import jax
import jax.numpy as jnp
from jax import lax
from jax.experimental import pallas as pl
from jax.experimental.pallas import tpu as pltpu
from jax.experimental.pallas import tpu_sc as plsc

BATCH = 16384
NNODE = 1000000
NF = 64
NC, NS, LANES = 2, 16, 16
NW = NC * NS
BPW = BATCH // NW
CHUNK = 128
NCH = BPW // CHUNK
GRP = BPW // LANES
BLK = 256
LASTB = (NNODE - 1) // BLK
LASTW = NNODE - LASTB * BLK
SENT = 2 ** 30


def _scan_table(wid, idx_hbm, tab_hbm, bias_hbm, tail_hbm, btail_hbm,
                rows_hbm, sb_hbm, si_v, blk0_v, blk1_v, blk2_v, blk3_v, blk4_v,
                bb0_v, bb1_v, bb2_v, bb3_v, bb4_v, st_v, sb_v,
                sem0, sem1, sem2, sem3, sem4):
  pltpu.sync_copy(idx_hbm.at[wid], si_v.at[pl.ds(0, BPW)])
  si_v[pl.ds(BPW, LANES)] = jnp.full((LANES,), SENT, jnp.int32)

  b_lo = si_v[pl.ds(0, LANES)][0] // BLK
  b_hi = si_v[pl.ds(BPW - LANES, LANES)][LANES - 1] // BLK
  nb = b_hi - b_lo + 1

  lanes = lax.iota(jnp.int32, LANES)
  rows = [lanes + k * LANES for k in range(NF // LANES)]
  lane0 = lanes == 0

  def fire(b, blk_v, bb_v, sem):
    @pl.when(jnp.logical_and(b <= b_hi, b < LASTB))
    def _():
      start = pl.multiple_of(b * BLK, BLK)
      pltpu.async_copy(tab_hbm.at[:, pl.ds(start, BLK)], blk_v, sem)
      pltpu.async_copy(bias_hbm.at[0, pl.ds(start, BLK)], bb_v, sem)

    @pl.when(jnp.logical_and(b <= b_hi, b == LASTB))
    def _():
      pltpu.async_copy(tail_hbm, blk_v, sem)
      pltpu.async_copy(btail_hbm, bb_v, sem)

  def wait(blk_v, bb_v, sem):
    pltpu.make_async_copy(tab_hbm.at[:, pl.ds(0, BLK)], blk_v, sem).wait()
    pltpu.make_async_copy(bias_hbm.at[0, pl.ds(0, BLK)], bb_v, sem).wait()

  def serve(b, blk_v, bb_v, c0):
    start = b * BLK

    def cond(c):
      w = si_v[pl.ds(c, LANES)]
      return jnp.logical_and(c < BPW, w[0] // BLK == b)

    def body(c):
      w = si_v[pl.ds(c, LANES)]
      l = jnp.full((LANES,), w[0] - start, jnp.int32)
      for k in range(NF // LANES):
        col = plsc.load_gather(blk_v, [rows[k], l])
        plsc.store_scatter(st_v, [c * NF + k * LANES + lanes], col)
      bv = plsc.load_gather(bb_v, [l])
      plsc.store_scatter(sb_v, [jnp.full((LANES,), c, jnp.int32)], bv,
                         mask=lane0)
      return c + 1

    return lax.while_loop(cond, body, c0)

  fire(b_lo, blk0_v, bb0_v, sem0)
  fire(b_lo + 1, blk1_v, bb1_v, sem1)
  fire(b_lo + 2, blk2_v, bb2_v, sem2)
  fire(b_lo + 3, blk3_v, bb3_v, sem3)
  fire(b_lo + 4, blk4_v, bb4_v, sem4)

  def step(b, blk_v, bb_v, sem, cc):
    def go(c):
      wait(blk_v, bb_v, sem)
      c = serve(b, blk_v, bb_v, c)
      fire(b + 5, blk_v, bb_v, sem)
      return c
    return lax.cond(b <= b_hi, go, lambda c: c, cc)

  def biter(m, c):
    b0 = b_lo + 5 * m
    c = step(b0, blk0_v, bb0_v, sem0, c)
    c = step(b0 + 1, blk1_v, bb1_v, sem1, c)
    c = step(b0 + 2, blk2_v, bb2_v, sem2, c)
    c = step(b0 + 3, blk3_v, bb3_v, sem3, c)
    c = step(b0 + 4, blk4_v, bb4_v, sem4, c)
    return c

  lax.fori_loop(0, (nb + 4) // 5, biter, 0)
  pltpu.sync_copy(st_v, rows_hbm.at[pl.ds(wid * BPW * NF, BPW * NF)])
  pltpu.sync_copy(sb_v, sb_hbm.at[pl.ds(wid * BPW, BPW)])


def _scan_body(siw_hbm, sio_hbm, wt_hbm, ht_hbm, wbt_hbm, hbt_hbm,
               wtail_hbm, htail_hbm, wbtail_hbm, hbtail_hbm,
               sw_hbm, sh_hbm, sbw_hbm, sbh_hbm,
               si_v, blk0_v, blk1_v, blk2_v, blk3_v, blk4_v,
               bb0_v, bb1_v, bb2_v, bb3_v, bb4_v, st_v, sb_v,
               sem0, sem1, sem2, sem3, sem4):
  wid = lax.axis_index("s") * NC + lax.axis_index("c")
  _scan_table(wid, siw_hbm, wt_hbm, wbt_hbm, wtail_hbm, wbtail_hbm,
              sw_hbm, sbw_hbm, si_v, blk0_v, blk1_v, blk2_v, blk3_v, blk4_v,
              bb0_v, bb1_v, bb2_v, bb3_v, bb4_v, st_v, sb_v,
              sem0, sem1, sem2, sem3, sem4)
  _scan_table(wid, sio_hbm, ht_hbm, hbt_hbm, htail_hbm, hbtail_hbm,
              sh_hbm, sbh_hbm, si_v, blk0_v, blk1_v, blk2_v, blk3_v, blk4_v,
              bb0_v, bb1_v, bb2_v, bb3_v, bb4_v, st_v, sb_v,
              sem0, sem1, sem2, sem3, sem4)


def _dot_body(rw_hbm, rh_hbm, sw_hbm, sh_hbm, sbw_hbm, sbh_hbm, out_hbm,
              rw_v, rh_v, bw_v, bh_v, cw_v, ch_v, o_v, sem0, sem1, semb):
  wid = lax.axis_index("s") * NC + lax.axis_index("c")

  pltpu.sync_copy(rw_hbm.at[wid], rw_v)
  pltpu.sync_copy(rh_hbm.at[wid], rh_v)

  for j in range(NCH):
    sl = pl.ds(j * CHUNK, CHUNK)
    pltpu.async_copy(sbw_hbm.at[rw_v.at[sl]], bw_v.at[sl], semb)
    pltpu.async_copy(sbh_hbm.at[rh_v.at[sl]], bh_v.at[sl], semb)

  lanes = lax.iota(jnp.int32, LANES)

  def fire(g, slot):
    ivw = rw_v[pl.ds(g * LANES, LANES)]
    ivh = rh_v[pl.ds(g * LANES, LANES)]
    sem = sem0 if slot == 0 else sem1
    for i in range(LANES):
      pltpu.async_copy(sw_hbm.at[pl.ds(ivw[i] * NF, NF)], cw_v.at[slot, i], sem)
      pltpu.async_copy(sh_hbm.at[pl.ds(ivh[i] * NF, NF)], ch_v.at[slot, i], sem)

  def drain(slot):
    sem = sem0 if slot == 0 else sem1
    for i in range(LANES):
      pltpu.make_async_copy(sw_hbm.at[pl.ds(0, NF)], cw_v.at[slot, i], sem).wait()
      pltpu.make_async_copy(sh_hbm.at[pl.ds(0, NF)], ch_v.at[slot, i], sem).wait()

  def compute(g, slot):
    acc = jnp.zeros((LANES,), jnp.float32)
    for i in range(LANES):
      s = cw_v[slot, i, pl.ds(0, LANES)] * ch_v[slot, i, pl.ds(0, LANES)]
      for k in range(1, NF // LANES):
        s = s + (cw_v[slot, i, pl.ds(k * LANES, LANES)] *
                 ch_v[slot, i, pl.ds(k * LANES, LANES)])
      acc = jnp.where(lanes == i, jnp.sum(s), acc)
    base = pl.ds(g * LANES, LANES)
    o_v[base] = acc + bw_v[base] + bh_v[base]

  pltpu.make_async_copy(sbw_hbm.at[pl.ds(0, BPW)], bw_v, semb).wait()
  pltpu.make_async_copy(sbh_hbm.at[pl.ds(0, BPW)], bh_v, semb).wait()

  fire(0, 0)
  fire(1, 1)

  def body(m, carry):
    g0 = 2 * m
    drain(0)
    compute(g0, 0)
    fire(g0 + 2, 0)
    drain(1)
    compute(g0 + 1, 1)
    fire(g0 + 3, 1)
    return carry

  lax.fori_loop(0, GRP // 2 - 1, body, 0)
  drain(0)
  compute(GRP - 2, 0)
  drain(1)
  compute(GRP - 1, 1)

  pltpu.sync_copy(o_v, out_hbm.at[pl.ds(wid * BPW, BPW)])


def kernel(nodes, W, H, w_bias, h_bias):
  nodes = nodes.astype(jnp.int32)
  iw = nodes[:, 0]
  io = nodes[:, 1]

  pos = jnp.arange(BATCH, dtype=jnp.int32)
  siw = jnp.sort(iw)
  sio = jnp.sort(io)
  rank_w = jnp.zeros((BATCH,), jnp.int32).at[jnp.argsort(iw)].set(pos)
  rank_h = jnp.zeros((BATCH,), jnp.int32).at[jnp.argsort(io)].set(pos)

  mesh = plsc.VectorSubcoreMesh(core_axis_name="c", subcore_axis_name="s",
                                num_cores=NC, num_subcores=NS)
  cp = pltpu.CompilerParams(needs_layout_passes=False)

  scan = pl.kernel(
      _scan_body,
      out_type=(jax.ShapeDtypeStruct((BATCH * NF,), jnp.float32),
                jax.ShapeDtypeStruct((BATCH * NF,), jnp.float32),
                jax.ShapeDtypeStruct((BATCH,), jnp.float32),
                jax.ShapeDtypeStruct((BATCH,), jnp.float32)),
      mesh=mesh,
      compiler_params=cp,
      scratch_types=[
          pltpu.VMEM((BPW + LANES,), jnp.int32),
          pltpu.VMEM((NF, BLK), jnp.float32),
          pltpu.VMEM((NF, BLK), jnp.float32),
          pltpu.VMEM((NF, BLK), jnp.float32),
          pltpu.VMEM((NF, BLK), jnp.float32),
          pltpu.VMEM((NF, BLK), jnp.float32),
          pltpu.VMEM((BLK,), jnp.float32),
          pltpu.VMEM((BLK,), jnp.float32),
          pltpu.VMEM((BLK,), jnp.float32),
          pltpu.VMEM((BLK,), jnp.float32),
          pltpu.VMEM((BLK,), jnp.float32),
          pltpu.VMEM((BPW * NF,), jnp.float32),
          pltpu.VMEM((BPW,), jnp.float32),
          pltpu.SemaphoreType.DMA,
          pltpu.SemaphoreType.DMA,
          pltpu.SemaphoreType.DMA,
          pltpu.SemaphoreType.DMA,
          pltpu.SemaphoreType.DMA,
      ],
  )
  pad = ((0, 0), (0, BLK - LASTW))
  wtail = jnp.pad(W[LASTB * BLK:].T, pad)
  htail = jnp.pad(H[LASTB * BLK:].T, pad)
  wbtail = jnp.pad(w_bias[LASTB * BLK:, 0], (0, BLK - LASTW))
  hbtail = jnp.pad(h_bias[LASTB * BLK:, 0], (0, BLK - LASTW))
  sw, sh, sbw, sbh = scan(siw.reshape(NW, BPW), sio.reshape(NW, BPW),
                          W.T, H.T, w_bias.T, h_bias.T,
                          wtail, htail, wbtail, hbtail)

  dot = pl.kernel(
      _dot_body,
      out_type=jax.ShapeDtypeStruct((BATCH,), jnp.float32),
      mesh=mesh,
      compiler_params=cp,
      scratch_types=[
          pltpu.VMEM((BPW,), jnp.int32),
          pltpu.VMEM((BPW,), jnp.int32),
          pltpu.VMEM((BPW,), jnp.float32),
          pltpu.VMEM((BPW,), jnp.float32),
          pltpu.VMEM((2, LANES, NF), jnp.float32),
          pltpu.VMEM((2, LANES, NF), jnp.float32),
          pltpu.VMEM((BPW,), jnp.float32),
          pltpu.SemaphoreType.DMA,
          pltpu.SemaphoreType.DMA,
          pltpu.SemaphoreType.DMA,
      ],
  )
  return dot(rank_w.reshape(NW, BPW), rank_h.reshape(NW, BPW),
             sw, sh, sbw, sbh)
